# Initial kernel scaffold; baseline (speedup 1.0000x reference)
import jax
import jax.numpy as jnp
from jax.experimental import pallas as pl


def kernel(x, w, b, gamma, beta):
    raise NotImplementedError("write your pallas kernel here")


# speedup vs baseline: 2.7323x; 2.7323x over previous
import functools

import jax
import jax.numpy as jnp
from jax import lax
from jax.experimental import pallas as pl
from jax.experimental.pallas import tpu as pltpu

EPS = 1e-5
NEG_SLOPE = 0.01


def _fused_kernel(n_chunks, rows_per_chunk, rows_per_step, nl,
                  x_hbm, w_ref, gamma_ref, beta_ref, o_ref,
                  x_vmem, scale_ref, shift_ref, sems):
    i = pl.program_id(0)
    cin = x_vmem.shape[1]

    @pl.when(i == 0)
    def _stats():
        for c in range(n_chunks):
            sl = pl.ds(c * rows_per_chunk, rows_per_chunk)
            pltpu.make_async_copy(x_hbm.at[sl], x_vmem.at[sl],
                                  sems.at[c]).start()
        g = jnp.zeros((cin, cin), jnp.float32)
        s = jnp.zeros((cin, 1), jnp.float32)
        for c in range(n_chunks):
            sl = pl.ds(c * rows_per_chunk, rows_per_chunk)
            pltpu.make_async_copy(x_hbm.at[sl], x_vmem.at[sl],
                                  sems.at[c]).wait()
            for r in range(rows_per_chunk):
                xn = x_vmem[c * rows_per_chunk + r]
                g += lax.dot_general(xn, xn, (((1,), (1,)), ((), ())),
                                     preferred_element_type=jnp.float32)
                s += jnp.sum(xn.astype(jnp.float32), axis=1, keepdims=True)
        inv_nl = jnp.float32(1.0 / nl)
        mean = s * inv_nl
        w32 = w_ref[...]
        mean_y = jnp.dot(w32, mean, preferred_element_type=jnp.float32)
        e_yy = jnp.sum(
            jnp.dot(w32, g * inv_nl, preferred_element_type=jnp.float32) * w32,
            axis=1, keepdims=True)
        var_y = e_yy - mean_y * mean_y
        inv_std = lax.rsqrt(var_y + EPS)
        scale = gamma_ref[...] * inv_std
        scale_ref[...] = scale
        shift_ref[...] = beta_ref[...] - mean_y * scale

    scale = scale_ref[...]
    shift = shift_ref[...]
    base = i * rows_per_step
    for r in range(rows_per_step):
        y = jnp.dot(w_ref[...], x_vmem[base + r],
                    preferred_element_type=jnp.float32)
        z = y * scale + shift
        o_ref[r] = jnp.maximum(z, NEG_SLOPE * z).astype(o_ref.dtype)


def kernel(x, w, b, gamma, beta):
    del b
    N, Cin, L = x.shape
    Cout = w.shape[0]

    rows_per_chunk = next(c for c in (8, 4, 2, 1) if N % c == 0)
    n_chunks = N // rows_per_chunk
    rows_per_step = next(c for c in (4, 2, 1) if N % c == 0)
    n_steps = N // rows_per_step

    w32 = w.astype(jnp.float32)
    gamma2 = gamma.astype(jnp.float32).reshape(Cout, 1)
    beta2 = beta.astype(jnp.float32).reshape(Cout, 1)

    body = functools.partial(_fused_kernel, n_chunks, rows_per_chunk,
                             rows_per_step, N * L)
    return pl.pallas_call(
        body,
        out_shape=jax.ShapeDtypeStruct((N, Cout, L), x.dtype),
        grid=(n_steps,),
        in_specs=[
            pl.BlockSpec(memory_space=pl.ANY),
            pl.BlockSpec((Cout, Cin), lambda i: (0, 0)),
            pl.BlockSpec((Cout, 1), lambda i: (0, 0)),
            pl.BlockSpec((Cout, 1), lambda i: (0, 0)),
        ],
        out_specs=pl.BlockSpec((rows_per_step, Cout, L), lambda i: (i, 0, 0)),
        scratch_shapes=[
            pltpu.VMEM((N, Cin, L), x.dtype),
            pltpu.VMEM((Cout, 1), jnp.float32),
            pltpu.VMEM((Cout, 1), jnp.float32),
            pltpu.SemaphoreType.DMA((n_chunks,)),
        ],
        compiler_params=pltpu.CompilerParams(
            dimension_semantics=("arbitrary",),
            vmem_limit_bytes=int(58 << 20),
        ),
    )(x, w32, gamma2, beta2)


# speedup vs baseline: 2.8294x; 1.0356x over previous
import functools

import jax
import jax.numpy as jnp
from jax import lax
from jax.experimental import pallas as pl
from jax.experimental.pallas import tpu as pltpu

EPS = 1e-5
NEG_SLOPE = 0.01


def _fused_kernel(n_chunks, rows_per_chunk, rows_per_step, nl,
                  x_hbm, w_ref, gamma_ref, beta_ref, o_ref,
                  x_vmem, ws_ref, shift_ref, sems):
    i = pl.program_id(0)
    cin = x_vmem.shape[1]

    @pl.when(i == 0)
    def _stats():
        for c in range(n_chunks):
            sl = pl.ds(c * rows_per_chunk, rows_per_chunk)
            pltpu.make_async_copy(x_hbm.at[sl], x_vmem.at[sl],
                                  sems.at[c]).start()
        g = jnp.zeros((cin, cin), jnp.float32)
        s = jnp.zeros((cin, 1), jnp.float32)
        for c in range(n_chunks):
            sl = pl.ds(c * rows_per_chunk, rows_per_chunk)
            pltpu.make_async_copy(x_hbm.at[sl], x_vmem.at[sl],
                                  sems.at[c]).wait()
            for r in range(rows_per_chunk):
                xn = x_vmem[c * rows_per_chunk + r]
                g += lax.dot_general(xn, xn, (((1,), (1,)), ((), ())),
                                     preferred_element_type=jnp.float32)
                s += jnp.sum(xn.astype(jnp.float32), axis=1, keepdims=True)
        inv_nl = jnp.float32(1.0 / nl)
        mean = s * inv_nl
        w32 = w_ref[...]
        mean_y = jnp.dot(w32, mean, preferred_element_type=jnp.float32)
        e_yy = jnp.sum(
            jnp.dot(w32, g * inv_nl, preferred_element_type=jnp.float32) * w32,
            axis=1, keepdims=True)
        var_y = e_yy - mean_y * mean_y
        inv_std = lax.rsqrt(var_y + EPS)
        scale = gamma_ref[...] * inv_std
        ws_ref[...] = w32 * scale
        shift_ref[...] = beta_ref[...] - mean_y * scale

    shift = shift_ref[...]
    ws = ws_ref[...]
    base = i * rows_per_step
    for r in range(rows_per_step):
        z = jnp.dot(ws, x_vmem[base + r],
                    preferred_element_type=jnp.float32) + shift
        o_ref[r] = jnp.maximum(z, NEG_SLOPE * z).astype(o_ref.dtype)


def kernel(x, w, b, gamma, beta):
    del b
    N, Cin, L = x.shape
    Cout = w.shape[0]

    rows_per_chunk = next(c for c in (8, 4, 2, 1) if N % c == 0)
    n_chunks = N // rows_per_chunk
    rows_per_step = next(c for c in (8, 4, 2, 1) if N % c == 0)
    n_steps = N // rows_per_step

    w32 = w.astype(jnp.float32)
    gamma2 = gamma.astype(jnp.float32).reshape(Cout, 1)
    beta2 = beta.astype(jnp.float32).reshape(Cout, 1)

    body = functools.partial(_fused_kernel, n_chunks, rows_per_chunk,
                             rows_per_step, N * L)
    return pl.pallas_call(
        body,
        out_shape=jax.ShapeDtypeStruct((N, Cout, L), x.dtype),
        grid=(n_steps,),
        in_specs=[
            pl.BlockSpec(memory_space=pl.ANY),
            pl.BlockSpec((Cout, Cin), lambda i: (0, 0)),
            pl.BlockSpec((Cout, 1), lambda i: (0, 0)),
            pl.BlockSpec((Cout, 1), lambda i: (0, 0)),
        ],
        out_specs=pl.BlockSpec((rows_per_step, Cout, L), lambda i: (i, 0, 0)),
        scratch_shapes=[
            pltpu.VMEM((N, Cin, L), x.dtype),
            pltpu.VMEM((Cout, Cin), jnp.float32),
            pltpu.VMEM((Cout, 1), jnp.float32),
            pltpu.SemaphoreType.DMA((n_chunks,)),
        ],
        compiler_params=pltpu.CompilerParams(
            dimension_semantics=("arbitrary",),
            vmem_limit_bytes=int(58 << 20),
        ),
    )(x, w32, gamma2, beta2)
